# Initial kernel scaffold; baseline (speedup 1.0000x reference)
import jax
import jax.numpy as jnp
from jax.experimental import pallas as pl


def kernel(x, pos, edge_index, edge_attr, batch, params):
    raise NotImplementedError("write your pallas kernel here")


# speedup vs baseline: 1.0170x; 1.0170x over previous
import functools

import jax
import jax.numpy as jnp
from jax.experimental import pallas as pl
from jax.experimental.pallas import tpu as pltpu


def _sig(v):
    return 1.0 / (1.0 + jnp.exp(-v))


def _silu(v):
    return v * _sig(v)


def _pick_block(n, target):
    b = min(n, target)
    while n % b:
        b -= 1
    return b


def _edge_body(hs_ref, hd_ref, geo_ref, ea_ref,
               whd_ref, whs_ref, wd_ref, we_ref, b1_ref,
               w2_ref, b2_ref, wa_ref, ba_ref,
               xhd_ref, xhs_ref, xd_ref, xe_ref, xb1_ref,
               xw2_ref, xb2_ref, xw3_ref, xb3_ref,
               msgh_ref, msgx_ref=None, *, last):
    hs = hs_ref[...]
    hd = hd_ref[...]
    geo = geo_ref[...]
    ea = ea_ref[...]
    d2 = geo[:, 0:1]

    pre = (hd @ whd_ref[...] + hs @ whs_ref[...] + d2 * wd_ref[...]
           + ea @ we_ref[...] + b1_ref[...])
    m = _silu(pre)
    m = _silu(m @ w2_ref[...] + b2_ref[...])
    attn = _sig(m @ wa_ref[...] + ba_ref[...])
    msgh_ref[...] = attn * m

    if not last:
        prex = (hd @ xhd_ref[...] + hs @ xhs_ref[...] + d2 * xd_ref[...]
                + ea @ xe_ref[...] + xb1_ref[...])
        mx = _silu(prex)
        mx = _silu(mx @ xw2_ref[...] + xb2_ref[...])
        mxs = mx @ xw3_ref[...] + xb3_ref[...]
        msgx_ref[...] = geo[:, 1:5] * mxs


def _edge_mlp(hs, hd, geo, ea, lp, last):
    E, F = hs.shape
    ED = ea.shape[1]
    BE = _pick_block(E, 2000)
    grid = (E // BE,)

    w1h = lp["msg_h"][0]["W"]
    w1x = lp["msg_x"][0]["W"]
    args = (
        hs, hd, geo, ea,
        w1h[:F], w1h[F:2 * F], w1h[2 * F:2 * F + 1], w1h[2 * F + 1:],
        lp["msg_h"][0]["b"][None, :],
        lp["msg_h"][1]["W"], lp["msg_h"][1]["b"][None, :],
        lp["attn"]["W"], lp["attn"]["b"][None, :],
        w1x[:F], w1x[F:2 * F], w1x[2 * F:2 * F + 1], w1x[2 * F + 1:],
        lp["msg_x"][0]["b"][None, :],
        lp["msg_x"][1]["W"], lp["msg_x"][1]["b"][None, :],
        lp["msg_x"][2]["W"], lp["msg_x"][2]["b"][None, :],
    )
    edge_spec = lambda c: pl.BlockSpec((BE, c), lambda i: (i, 0))
    full_spec = lambda a: pl.BlockSpec(a.shape, lambda i: (0,) * a.ndim)
    in_specs = [edge_spec(F), edge_spec(F), edge_spec(8), edge_spec(ED)]
    in_specs += [full_spec(a) for a in args[4:]]
    out_shape = [jax.ShapeDtypeStruct((E, F), jnp.float32),
                 jax.ShapeDtypeStruct((E, 4), jnp.float32)]
    out_specs = [edge_spec(F), edge_spec(4)]
    if last:
        out_shape = out_shape[:1]
        out_specs = out_specs[:1]
    out = pl.pallas_call(
        functools.partial(_edge_body, last=last),
        grid=grid,
        in_specs=in_specs,
        out_specs=out_specs,
        out_shape=out_shape,
    )(*args)
    return (out[0], out[1]) if not last else (out[0], None)


def _upd_body(h_ref, agg_ref, w1a_ref, w1b_ref, b1_ref, w2_ref, b2_ref, out_ref):
    h = h_ref[...]
    u = _silu(h @ w1a_ref[...] + agg_ref[...] @ w1b_ref[...] + b1_ref[...])
    out_ref[...] = h + (u @ w2_ref[...] + b2_ref[...])


def _node_update(h, agg_h, lp):
    N, F = h.shape
    BN = _pick_block(N, 2000)
    w1 = lp["upd"][0]["W"]
    args = (h, agg_h, w1[:F], w1[F:], lp["upd"][0]["b"][None, :],
            lp["upd"][1]["W"], lp["upd"][1]["b"][None, :])
    node_spec = pl.BlockSpec((BN, F), lambda i: (i, 0))
    full_spec = lambda a: pl.BlockSpec(a.shape, lambda i: (0,) * a.ndim)
    return pl.pallas_call(
        _upd_body,
        grid=(N // BN,),
        in_specs=[node_spec, node_spec] + [full_spec(a) for a in args[2:]],
        out_specs=node_spec,
        out_shape=jax.ShapeDtypeStruct((N, F), jnp.float32),
    )(*args)


def _pool_body(h_ref, b_ref, wp_ref, bp_ref, out_ref, sums_ref, cnt_ref, *, G, steps):
    i = pl.program_id(0)

    @pl.when(i == 0)
    def _init():
        sums_ref[...] = jnp.zeros_like(sums_ref)
        cnt_ref[...] = jnp.zeros_like(cnt_ref)

    h = h_ref[...]
    b = b_ref[...]
    BN = h.shape[0]
    onehot = (jax.lax.broadcasted_iota(jnp.int32, (BN, G), 1) == b).astype(jnp.float32)
    dn = (((0,), (0,)), ((), ()))
    sums_ref[...] += jax.lax.dot_general(onehot, h, dn)
    cnt_ref[...] += jax.lax.dot_general(onehot, jnp.ones_like(h), dn)

    @pl.when(i == steps - 1)
    def _fin():
        hg = sums_ref[...] / jnp.maximum(cnt_ref[...], 1.0)
        out_ref[...] = hg @ wp_ref[...] + bp_ref[...]


def _pool_head(h, batch, pp, G):
    N, F = h.shape
    BN = _pick_block(N, 2000)
    steps = N // BN
    args = (h, batch[:, None], pp["W"], pp["b"][None, :])
    full_spec = lambda a: pl.BlockSpec(a.shape, lambda i: (0,) * a.ndim)
    out = pl.pallas_call(
        functools.partial(_pool_body, G=G, steps=steps),
        grid=(steps,),
        in_specs=[pl.BlockSpec((BN, F), lambda i: (i, 0)),
                  pl.BlockSpec((BN, 1), lambda i: (i, 0)),
                  full_spec(args[2]), full_spec(args[3])],
        out_specs=pl.BlockSpec((G, 1), lambda i: (0, 0)),
        out_shape=jax.ShapeDtypeStruct((G, 1), jnp.float32),
        scratch_shapes=[pltpu.VMEM((G, F), jnp.float32),
                        pltpu.VMEM((G, F), jnp.float32)],
    )(*args)
    return out.reshape(-1)


def kernel(x, pos, edge_index, edge_attr, batch, params):
    h = x
    xp = pos
    src = edge_index[0]
    dst = edge_index[1]
    G = 64
    layers = params["layers"]
    L = len(layers)
    for li, lp in enumerate(layers):
        last = li == L - 1
        hs = h[src]
        hd = h[dst]
        xs = xp[src]
        xd = xp[dst]
        diff = xd - xs
        d2 = jnp.sum(diff * diff, axis=-1, keepdims=True) + 1e-12
        d = jnp.sqrt(d2)
        diffn = diff / (d + 1.0)
        geo = jnp.concatenate(
            [d2, diffn, jnp.zeros((d2.shape[0], 4), jnp.float32)], axis=1)
        msg_h, msg_x = _edge_mlp(hs, hd, geo, ea := edge_attr, lp, last)
        agg_h = jax.ops.segment_sum(msg_h, dst, num_segments=h.shape[0])
        h = _node_update(h, agg_h, lp)
        if not last:
            agg_x = jax.ops.segment_sum(msg_x, dst, num_segments=h.shape[0])
            xp = xp + agg_x[:, :3]
    return _pool_head(h, batch, params["pred"], G)


# speedup vs baseline: 1.5808x; 1.5543x over previous
import functools

import jax
import jax.numpy as jnp
from jax import lax
from jax.experimental import pallas as pl
from jax.experimental.pallas import tpu as pltpu
from jax.experimental.pallas import tpu_sc as plsc

_NC = 2
_NS = 16
_NW = _NC * _NS


def _sig(v):
    return 1.0 / (1.0 + jnp.exp(-v))


def _silu(v):
    return v * _sig(v)


def _pick_block(n, target):
    b = min(n, target)
    while n % b:
        b -= 1
    return b


_GB = 80
_GR = 5
_GD = 2


def _gather_body(tab_hbm, idx_hbm, out_hbm, idxv, bufs, gs, os):
    B, R, D = _GB, _GR, _GD
    per_w = idx_hbm.shape[0] // _NW
    nblk = per_w // B
    wid = lax.axis_index("s") * _NC + lax.axis_index("c")
    base = wid * per_w
    pltpu.sync_copy(idx_hbm.at[pl.ds(base, per_w)], idxv)

    def fire(b, j):
        ids = idxv.at[pl.ds(b * B, B)]
        pltpu.async_copy(tab_hbm.at[ids], bufs.at[j], gs.at[j])

    def wait_gather(j):
        ids0 = idxv.at[pl.ds(0, B)]
        pltpu.make_async_copy(tab_hbm.at[ids0], bufs.at[j], gs.at[j]).wait()

    def wait_out(j):
        pltpu.make_async_copy(bufs.at[j], out_hbm.at[pl.ds(0, B)], os.at[j]).wait()

    for j in range(R):
        fire(j, j)

    def group(g, carry):
        for j in range(R):
            b = g * R + j
            off = base + b * B
            wait_gather(j)
            pltpu.async_copy(bufs.at[j], out_hbm.at[pl.ds(off, B)], os.at[j])
            j2 = (j - D) % R
            bprev = b - D
            nxt = bprev + R

            @pl.when(jnp.logical_and(bprev >= 0, nxt < nblk))
            def _():
                wait_out(j2)
                fire(nxt, j2)
        return carry

    lax.fori_loop(0, nblk // R, group, 0)
    for j in range(R):
        wait_out(j)


def _sc_gather(tab, idx_all):
    M = idx_all.shape[0]
    F = tab.shape[1]
    per_w = M // _NW
    assert M % _NW == 0 and per_w % (_GB * _GR) == 0
    mesh = plsc.VectorSubcoreMesh(core_axis_name="c", subcore_axis_name="s",
                                  num_cores=_NC, num_subcores=_NS)
    k = pl.kernel(
        _gather_body,
        out_type=jax.ShapeDtypeStruct((M, F), jnp.float32),
        mesh=mesh,
        scratch_types=[
            pltpu.VMEM((per_w,), jnp.int32),
            pltpu.VMEM((_GR, _GB, F), jnp.float32),
            pltpu.SemaphoreType.DMA((_GR,)),
            pltpu.SemaphoreType.DMA((_GR,)),
        ],
    )
    return k(tab, idx_all)


def _edge_body(hs_ref, hd_ref, xs_ref, xd_ref, ea_ref,
               whd_ref, whs_ref, wd_ref, we_ref, b1_ref,
               w2_ref, b2_ref, wa_ref, ba_ref,
               xhd_ref, xhs_ref, xd2_ref, xe_ref, xb1_ref,
               xw2_ref, xb2_ref, xw3_ref, xb3_ref,
               msgh_ref, msgx_ref=None, *, last):
    hs = hs_ref[...]
    hd = hd_ref[...]
    ea = ea_ref[...]
    diff = xd_ref[...] - xs_ref[...]
    d2 = jnp.sum(diff * diff, axis=1, keepdims=True) + 1e-12

    pre = (hd @ whd_ref[...] + hs @ whs_ref[...] + d2 * wd_ref[...]
           + ea @ we_ref[...] + b1_ref[...])
    m = _silu(pre)
    m = _silu(m @ w2_ref[...] + b2_ref[...])
    attn = _sig(m @ wa_ref[...] + ba_ref[...])
    msgh_ref[...] = attn * m

    if not last:
        prex = (hd @ xhd_ref[...] + hs @ xhs_ref[...] + d2 * xd2_ref[...]
                + ea @ xe_ref[...] + xb1_ref[...])
        mx = _silu(prex)
        mx = _silu(mx @ xw2_ref[...] + xb2_ref[...])
        mxs = mx @ xw3_ref[...] + xb3_ref[...]
        d = jnp.sqrt(d2)
        xw = msgx_ref.shape[1]
        msgx_ref[...] = diff[:, :xw] / (d + 1.0) * mxs


def _edge_mlp(hs, hd, xs, xd, ea, lp, last):
    E, F = hs.shape
    ED = ea.shape[1]
    XW = 16
    BE = _pick_block(E, 2000)
    grid = (E // BE,)

    w1h = lp["msg_h"][0]["W"]
    w1x = lp["msg_x"][0]["W"]
    args = (
        hs, hd, xs, xd, ea,
        w1h[:F], w1h[F:2 * F], w1h[2 * F:2 * F + 1], w1h[2 * F + 1:],
        lp["msg_h"][0]["b"][None, :],
        lp["msg_h"][1]["W"], lp["msg_h"][1]["b"][None, :],
        lp["attn"]["W"], lp["attn"]["b"][None, :],
        w1x[:F], w1x[F:2 * F], w1x[2 * F:2 * F + 1], w1x[2 * F + 1:],
        lp["msg_x"][0]["b"][None, :],
        lp["msg_x"][1]["W"], lp["msg_x"][1]["b"][None, :],
        lp["msg_x"][2]["W"], lp["msg_x"][2]["b"][None, :],
    )
    edge_spec = lambda c: pl.BlockSpec((BE, c), lambda i: (i, 0))
    full_spec = lambda a: pl.BlockSpec(a.shape, lambda i: (0,) * a.ndim)
    in_specs = [edge_spec(F), edge_spec(F), edge_spec(F), edge_spec(F),
                edge_spec(ED)]
    in_specs += [full_spec(a) for a in args[5:]]
    out_shape = [jax.ShapeDtypeStruct((E, F), jnp.float32),
                 jax.ShapeDtypeStruct((E, XW), jnp.float32)]
    out_specs = [edge_spec(F), edge_spec(XW)]
    if last:
        out_shape = out_shape[:1]
        out_specs = out_specs[:1]
    out = pl.pallas_call(
        functools.partial(_edge_body, last=last),
        grid=grid,
        in_specs=in_specs,
        out_specs=out_specs,
        out_shape=out_shape,
    )(*args)
    return (out[0], out[1]) if not last else (out[0], None)


def _upd_body(h_ref, agg_ref, w1a_ref, w1b_ref, b1_ref, w2_ref, b2_ref, out_ref):
    h = h_ref[...]
    u = _silu(h @ w1a_ref[...] + agg_ref[...] @ w1b_ref[...] + b1_ref[...])
    out_ref[...] = h + (u @ w2_ref[...] + b2_ref[...])


def _node_update(h, agg_h, lp):
    N, F = h.shape
    BN = _pick_block(N, 2000)
    w1 = lp["upd"][0]["W"]
    args = (h, agg_h, w1[:F], w1[F:], lp["upd"][0]["b"][None, :],
            lp["upd"][1]["W"], lp["upd"][1]["b"][None, :])
    node_spec = pl.BlockSpec((BN, F), lambda i: (i, 0))
    full_spec = lambda a: pl.BlockSpec(a.shape, lambda i: (0,) * a.ndim)
    return pl.pallas_call(
        _upd_body,
        grid=(N // BN,),
        in_specs=[node_spec, node_spec] + [full_spec(a) for a in args[2:]],
        out_specs=node_spec,
        out_shape=jax.ShapeDtypeStruct((N, F), jnp.float32),
    )(*args)


def _pool_body(h_ref, b_ref, wp_ref, bp_ref, out_ref, sums_ref, cnt_ref, *, G, steps):
    i = pl.program_id(0)

    @pl.when(i == 0)
    def _init():
        sums_ref[...] = jnp.zeros_like(sums_ref)
        cnt_ref[...] = jnp.zeros_like(cnt_ref)

    h = h_ref[...]
    b = b_ref[...]
    BN = h.shape[0]
    onehot = (jax.lax.broadcasted_iota(jnp.int32, (BN, G), 1) == b).astype(jnp.float32)
    dn = (((0,), (0,)), ((), ()))
    sums_ref[...] += jax.lax.dot_general(onehot, h, dn)
    cnt_ref[...] += jax.lax.dot_general(onehot, jnp.ones_like(h), dn)

    @pl.when(i == steps - 1)
    def _fin():
        hg = sums_ref[...] / jnp.maximum(cnt_ref[...], 1.0)
        out_ref[...] = hg @ wp_ref[...] + bp_ref[...]


def _pool_head(h, batch, pp, G):
    N, F = h.shape
    BN = _pick_block(N, 2000)
    steps = N // BN
    args = (h, batch[:, None], pp["W"], pp["b"][None, :])
    full_spec = lambda a: pl.BlockSpec(a.shape, lambda i: (0,) * a.ndim)
    out = pl.pallas_call(
        functools.partial(_pool_body, G=G, steps=steps),
        grid=(steps,),
        in_specs=[pl.BlockSpec((BN, F), lambda i: (i, 0)),
                  pl.BlockSpec((BN, 1), lambda i: (i, 0)),
                  full_spec(args[2]), full_spec(args[3])],
        out_specs=pl.BlockSpec((G, 1), lambda i: (0, 0)),
        out_shape=jax.ShapeDtypeStruct((G, 1), jnp.float32),
        scratch_shapes=[pltpu.VMEM((G, F), jnp.float32),
                        pltpu.VMEM((G, F), jnp.float32)],
    )(*args)
    return out.reshape(-1)


def kernel(x, pos, edge_index, edge_attr, batch, params):
    h = x
    N, F = h.shape
    E = edge_index.shape[1]
    xq = jnp.concatenate([pos, jnp.zeros((N, F - 3), jnp.float32)], axis=1)
    idx_all = edge_index.reshape(-1)
    idx2 = jnp.concatenate([idx_all, idx_all + N])
    dst = edge_index[1]
    G = 64
    layers = params["layers"]
    L = len(layers)
    for li, lp in enumerate(layers):
        last = li == L - 1
        tab = jnp.concatenate([h, xq], axis=0)
        g = _sc_gather(tab, idx2)
        hs, hd = g[:E], g[E:2 * E]
        xs, xd = g[2 * E:3 * E], g[3 * E:]
        msg_h, msg_x = _edge_mlp(hs, hd, xs, xd, edge_attr, lp, last)
        agg_h = jax.ops.segment_sum(msg_h, dst, num_segments=N)
        h = _node_update(h, agg_h, lp)
        if not last:
            agg_x = jax.ops.segment_sum(msg_x, dst, num_segments=N)
            xq = xq + jnp.pad(agg_x, ((0, 0), (0, F - agg_x.shape[1])))
    return _pool_head(h, batch, params["pred"], G)
